# baseline (speedup 1.0000x reference)
import jax
import jax.numpy as jnp
from jax.experimental import pallas as pl
from jax.experimental.pallas import tpu as pltpu


def _bn(h, g, b):
    m = jnp.mean(h, axis=0, keepdims=True)
    v = jnp.mean((h - m) ** 2, axis=0, keepdims=True)
    return (h - m) * jax.lax.rsqrt(v + 1e-5) * g + b


def _fwd_kernel(x_ref, adj_ref, W_feat, Wc0, Wc1, Wc2, W_l0, W_cls, vec_ref,
                out_ref, h_ref):
    B, N, _ = adj_ref.shape
    F = x_ref.shape[1]
    C = W_cls.shape[1]

    def vrow(i, width):
        return vec_ref[i:i + 1, 0:width]

    dinv = jnp.concatenate(
        [jax.lax.rsqrt(jnp.sum(adj_ref[b], axis=1, keepdims=True) + 1.0)
         for b in range(B)], axis=0)

    h = _bn(x_ref[:], vrow(0, F), vrow(1, F))
    h_ref[:, :] = jnp.maximum(
        jnp.dot(h, W_feat[:], preferred_element_type=jnp.float32)
        + vrow(2, 192), 0.0)

    for (W, base) in ((Wc0, 3), (Wc1, 6), (Wc2, 9)):
        h = _bn(h_ref[:, :], vrow(base, 192), vrow(base + 1, 192))
        hw = jnp.dot(h, W[:], preferred_element_type=jnp.float32)
        v = dinv * hw
        for b in range(B):
            vb = v[b * N:(b + 1) * N]
            rb = jax.lax.dot_general(adj_ref[b], vb,
                                     (((0,), (0,)), ((), ())),
                                     preferred_element_type=jnp.float32)
            h_ref[b * N:(b + 1) * N, :] = jnp.maximum(
                dinv[b * N:(b + 1) * N] * (rb + vb) + vrow(base + 2, 192), 0.0)

    pooled = jnp.concatenate(
        [jnp.mean(h_ref[b * N:(b + 1) * N, :], axis=0, keepdims=True)
         for b in range(B)], axis=0)

    z = _bn(pooled, vrow(12, 192), vrow(13, 192))
    z = jnp.maximum(
        jnp.dot(z, W_l0[:], preferred_element_type=jnp.float32)
        + vrow(14, 192), 0.0)
    z = _bn(z, vrow(15, 192), vrow(16, 192))
    logits = (jnp.dot(z, W_cls[:], preferred_element_type=jnp.float32)
              + vrow(17, C))
    e = logits - jnp.max(logits, axis=1, keepdims=True)
    out_ref[:, :] = e - jnp.log(jnp.sum(jnp.exp(e), axis=1, keepdims=True))


def kernel(x, adj, bn_feat_g, bn_feat_b, W_feat, b_feat,
           bnc0_g, bnc0_b, Wc0, bc0,
           bnc1_g, bnc1_b, Wc1, bc1,
           bnc2_g, bnc2_b, Wc2, bc2,
           bnfc0_g, bnfc0_b, W_l0, b_l0,
           bn_h_g, bn_h_b, W_cls, b_cls):
    B, N, F = x.shape
    H = W_feat.shape[1]
    C = W_cls.shape[1]

    def pad_row(v):
        return jnp.pad(v, (0, 192 - v.shape[0]))

    vecs = jnp.stack([
        pad_row(bn_feat_g), pad_row(bn_feat_b), pad_row(b_feat),
        pad_row(bnc0_g), pad_row(bnc0_b), pad_row(bc0),
        pad_row(bnc1_g), pad_row(bnc1_b), pad_row(bc1),
        pad_row(bnc2_g), pad_row(bnc2_b), pad_row(bc2),
        pad_row(bnfc0_g), pad_row(bnfc0_b), pad_row(b_l0),
        pad_row(bn_h_g), pad_row(bn_h_b), pad_row(b_cls),
    ])

    return pl.pallas_call(
        _fwd_kernel,
        out_shape=jax.ShapeDtypeStruct((B, C), jnp.float32),
        scratch_shapes=[pltpu.VMEM((B * N, H), jnp.float32)],
    )(x.reshape(B * N, F), adj, W_feat, Wc0, Wc1, Wc2, W_l0, W_cls, vecs)


# speedup vs baseline: 1.4764x; 1.4764x over previous
import jax
import jax.numpy as jnp
from jax.experimental import pallas as pl
from jax.experimental.pallas import tpu as pltpu


def _bn(h, g, b):
    m = jnp.mean(h, axis=0, keepdims=True)
    v = jnp.mean((h - m) ** 2, axis=0, keepdims=True)
    return (h - m) * jax.lax.rsqrt(v + 1e-5) * g + b


def _fwd_kernel(x_ref, adj_hbm, bn_feat_g, bn_feat_b, W_feat, b_feat,
                bnc0_g, bnc0_b, Wc0, bc0,
                bnc1_g, bnc1_b, Wc1, bc1,
                bnc2_g, bnc2_b, Wc2, bc2,
                bnfc0_g, bnfc0_b, W_l0, b_l0,
                bn_h_g, bn_h_b, W_cls, b_cls,
                out_ref, h_ref, adj_ref, sem):
    B, N, _ = adj_ref.shape

    adj_copy = pltpu.make_async_copy(adj_hbm, adj_ref, sem)
    adj_copy.start()

    h = _bn(x_ref[:], bn_feat_g[:], bn_feat_b[:])
    h_ref[:, :] = jnp.maximum(
        jnp.dot(h, W_feat[:], preferred_element_type=jnp.float32) + b_feat[:],
        0.0)

    adj_copy.wait()

    dinv = jnp.concatenate(
        [jax.lax.rsqrt(jnp.sum(adj_ref[b], axis=1, keepdims=True) + 1.0)
         for b in range(B)], axis=0)

    for (g, bb, W, bias) in ((bnc0_g, bnc0_b, Wc0, bc0),
                             (bnc1_g, bnc1_b, Wc1, bc1),
                             (bnc2_g, bnc2_b, Wc2, bc2)):
        h = _bn(h_ref[:, :], g[:], bb[:])
        hw = jnp.dot(h, W[:], preferred_element_type=jnp.float32)
        v = dinv * hw
        for b in range(B):
            vb = v[b * N:(b + 1) * N]
            rb = jax.lax.dot_general(adj_ref[b], vb,
                                     (((0,), (0,)), ((), ())),
                                     preferred_element_type=jnp.float32)
            h_ref[b * N:(b + 1) * N, :] = jnp.maximum(
                dinv[b * N:(b + 1) * N] * (rb + vb) + bias[:], 0.0)

    pooled = jnp.concatenate(
        [jnp.mean(h_ref[b * N:(b + 1) * N, :], axis=0, keepdims=True)
         for b in range(B)], axis=0)

    z = _bn(pooled, bnfc0_g[:], bnfc0_b[:])
    z = jnp.maximum(
        jnp.dot(z, W_l0[:], preferred_element_type=jnp.float32) + b_l0[:], 0.0)
    z = _bn(z, bn_h_g[:], bn_h_b[:])
    logits = jnp.dot(z, W_cls[:], preferred_element_type=jnp.float32) + b_cls[:]
    e = logits - jnp.max(logits, axis=1, keepdims=True)
    out_ref[:, :] = e - jnp.log(jnp.sum(jnp.exp(e), axis=1, keepdims=True))


def kernel(x, adj, bn_feat_g, bn_feat_b, W_feat, b_feat,
           bnc0_g, bnc0_b, Wc0, bc0,
           bnc1_g, bnc1_b, Wc1, bc1,
           bnc2_g, bnc2_b, Wc2, bc2,
           bnfc0_g, bnfc0_b, W_l0, b_l0,
           bn_h_g, bn_h_b, W_cls, b_cls):
    B, N, F = x.shape
    H = W_feat.shape[1]
    C = W_cls.shape[1]
    row = lambda a: a.reshape(1, -1)
    n_in = 26
    specs = [pl.BlockSpec(memory_space=pl.MemorySpace.ANY) if i == 1
             else pl.BlockSpec(memory_space=pltpu.MemorySpace.VMEM)
             for i in range(n_in)]
    return pl.pallas_call(
        _fwd_kernel,
        out_shape=jax.ShapeDtypeStruct((B, C), jnp.float32),
        in_specs=specs,
        scratch_shapes=[pltpu.VMEM((B * N, H), jnp.float32),
                        pltpu.VMEM((B, N, N), jnp.float32),
                        pltpu.SemaphoreType.DMA],
    )(x.reshape(B * N, F), adj,
      row(bn_feat_g), row(bn_feat_b), W_feat, row(b_feat),
      row(bnc0_g), row(bnc0_b), Wc0, row(bc0),
      row(bnc1_g), row(bnc1_b), Wc1, row(bc1),
      row(bnc2_g), row(bnc2_b), Wc2, row(bc2),
      row(bnfc0_g), row(bnfc0_b), W_l0, row(b_l0),
      row(bn_h_g), row(bn_h_b), W_cls, row(b_cls))


# speedup vs baseline: 1.5264x; 1.0339x over previous
import jax
import jax.numpy as jnp
from jax.experimental import pallas as pl
from jax.experimental.pallas import tpu as pltpu


def _bn(h, g, b):
    m = jnp.mean(h, axis=0, keepdims=True)
    sq = jnp.mean(h * h, axis=0, keepdims=True)
    a = jax.lax.rsqrt(jnp.maximum(sq - m * m, 0.0) + 1e-5) * g
    return h * a + (b - m * a)


def _fwd_kernel(x_ref, adj_hbm, bn_feat_g, bn_feat_b, W_feat, b_feat,
                bnc0_g, bnc0_b, Wc0, bc0,
                bnc1_g, bnc1_b, Wc1, bc1,
                bnc2_g, bnc2_b, Wc2, bc2,
                bnfc0_g, bnfc0_b, W_l0, b_l0,
                bn_h_g, bn_h_b, W_cls, b_cls,
                out_ref, h_ref, adj_ref, sem):
    B, N, _ = adj_ref.shape

    adj_copy = pltpu.make_async_copy(adj_hbm, adj_ref, sem)
    adj_copy.start()

    h = _bn(x_ref[:], bn_feat_g[:], bn_feat_b[:])
    h_ref[:, :] = jnp.maximum(
        jnp.dot(h, W_feat[:], preferred_element_type=jnp.float32) + b_feat[:],
        0.0)

    adj_copy.wait()

    dinv = jnp.concatenate(
        [jax.lax.rsqrt(jnp.sum(adj_ref[b], axis=1, keepdims=True) + 1.0)
         for b in range(B)], axis=0)

    for (g, bb, W, bias) in ((bnc0_g, bnc0_b, Wc0, bc0),
                             (bnc1_g, bnc1_b, Wc1, bc1),
                             (bnc2_g, bnc2_b, Wc2, bc2)):
        h = _bn(h_ref[:, :], g[:], bb[:])
        hw = jnp.dot(h, W[:], preferred_element_type=jnp.float32)
        v = dinv * hw
        for b in range(B):
            vb = v[b * N:(b + 1) * N]
            rb = jax.lax.dot_general(adj_ref[b], vb,
                                     (((0,), (0,)), ((), ())),
                                     preferred_element_type=jnp.float32)
            h_ref[b * N:(b + 1) * N, :] = jnp.maximum(
                dinv[b * N:(b + 1) * N] * (rb + vb) + bias[:], 0.0)

    pooled = jnp.concatenate(
        [jnp.mean(h_ref[b * N:(b + 1) * N, :], axis=0, keepdims=True)
         for b in range(B)], axis=0)

    z = _bn(pooled, bnfc0_g[:], bnfc0_b[:])
    z = jnp.maximum(
        jnp.dot(z, W_l0[:], preferred_element_type=jnp.float32) + b_l0[:], 0.0)
    z = _bn(z, bn_h_g[:], bn_h_b[:])
    logits = jnp.dot(z, W_cls[:], preferred_element_type=jnp.float32) + b_cls[:]
    e = logits - jnp.max(logits, axis=1, keepdims=True)
    out_ref[:, :] = e - jnp.log(jnp.sum(jnp.exp(e), axis=1, keepdims=True))


def kernel(x, adj, bn_feat_g, bn_feat_b, W_feat, b_feat,
           bnc0_g, bnc0_b, Wc0, bc0,
           bnc1_g, bnc1_b, Wc1, bc1,
           bnc2_g, bnc2_b, Wc2, bc2,
           bnfc0_g, bnfc0_b, W_l0, b_l0,
           bn_h_g, bn_h_b, W_cls, b_cls):
    B, N, F = x.shape
    H = W_feat.shape[1]
    C = W_cls.shape[1]
    row = lambda a: a.reshape(1, -1)
    n_in = 26
    specs = [pl.BlockSpec(memory_space=pl.MemorySpace.ANY) if i == 1
             else pl.BlockSpec(memory_space=pltpu.MemorySpace.VMEM)
             for i in range(n_in)]
    return pl.pallas_call(
        _fwd_kernel,
        out_shape=jax.ShapeDtypeStruct((B, C), jnp.float32),
        in_specs=specs,
        scratch_shapes=[pltpu.VMEM((B * N, H), jnp.float32),
                        pltpu.VMEM((B, N, N), jnp.float32),
                        pltpu.SemaphoreType.DMA],
    )(x.reshape(B * N, F), adj,
      row(bn_feat_g), row(bn_feat_b), W_feat, row(b_feat),
      row(bnc0_g), row(bnc0_b), Wc0, row(bc0),
      row(bnc1_g), row(bnc1_b), Wc1, row(bc1),
      row(bnc2_g), row(bnc2_b), Wc2, row(bc2),
      row(bnfc0_g), row(bnfc0_b), W_l0, row(b_l0),
      row(bn_h_g), row(bn_h_b), W_cls, row(b_cls))


# speedup vs baseline: 1.5344x; 1.0053x over previous
import jax
import jax.numpy as jnp
from jax.experimental import pallas as pl
from jax.experimental.pallas import tpu as pltpu


def _bn(h, g, b):
    n = h.shape[0]
    if n >= 256:
        ones_row = jnp.full((1, n), 1.0 / n, dtype=jnp.float32)
        m = jnp.dot(ones_row, h, preferred_element_type=jnp.float32)
        sq = jnp.dot(ones_row, h * h, preferred_element_type=jnp.float32)
    else:
        m = jnp.mean(h, axis=0, keepdims=True)
        sq = jnp.mean(h * h, axis=0, keepdims=True)
    a = jax.lax.rsqrt(jnp.maximum(sq - m * m, 0.0) + 1e-5) * g
    return h * a + (b - m * a)


def _fwd_kernel(x_ref, adj_hbm, bn_feat_g, bn_feat_b, W_feat, b_feat,
                bnc0_g, bnc0_b, Wc0, bc0,
                bnc1_g, bnc1_b, Wc1, bc1,
                bnc2_g, bnc2_b, Wc2, bc2,
                bnfc0_g, bnfc0_b, W_l0, b_l0,
                bn_h_g, bn_h_b, W_cls, b_cls,
                out_ref, h_ref, adj_ref, sem):
    B, N, _ = adj_ref.shape

    adj_copy = pltpu.make_async_copy(adj_hbm, adj_ref, sem)
    adj_copy.start()

    h = _bn(x_ref[:], bn_feat_g[:], bn_feat_b[:])
    h_ref[:, :] = jnp.maximum(
        jnp.dot(h, W_feat[:], preferred_element_type=jnp.float32) + b_feat[:],
        0.0)

    adj_copy.wait()

    dinv = jnp.concatenate(
        [jax.lax.rsqrt(jnp.sum(adj_ref[b], axis=1, keepdims=True) + 1.0)
         for b in range(B)], axis=0)

    for (g, bb, W, bias) in ((bnc0_g, bnc0_b, Wc0, bc0),
                             (bnc1_g, bnc1_b, Wc1, bc1),
                             (bnc2_g, bnc2_b, Wc2, bc2)):
        h = _bn(h_ref[:, :], g[:], bb[:])
        hw = jnp.dot(h, W[:], preferred_element_type=jnp.float32)
        v = dinv * hw
        for b in range(B):
            vb = v[b * N:(b + 1) * N]
            rb = jax.lax.dot_general(adj_ref[b], vb,
                                     (((0,), (0,)), ((), ())),
                                     preferred_element_type=jnp.float32)
            h_ref[b * N:(b + 1) * N, :] = jnp.maximum(
                dinv[b * N:(b + 1) * N] * (rb + vb) + bias[:], 0.0)

    gi = jax.lax.broadcasted_iota(jnp.int32, (B, B * N), 0)
    ni = jax.lax.broadcasted_iota(jnp.int32, (B, B * N), 1)
    pool_w = jnp.where(ni // N == gi, 1.0 / N, 0.0)
    pooled = jnp.dot(pool_w, h_ref[:, :],
                     preferred_element_type=jnp.float32)

    z = _bn(pooled, bnfc0_g[:], bnfc0_b[:])
    z = jnp.maximum(
        jnp.dot(z, W_l0[:], preferred_element_type=jnp.float32) + b_l0[:], 0.0)
    z = _bn(z, bn_h_g[:], bn_h_b[:])
    logits = jnp.dot(z, W_cls[:], preferred_element_type=jnp.float32) + b_cls[:]
    e = logits - jnp.max(logits, axis=1, keepdims=True)
    out_ref[:, :] = e - jnp.log(jnp.sum(jnp.exp(e), axis=1, keepdims=True))


def kernel(x, adj, bn_feat_g, bn_feat_b, W_feat, b_feat,
           bnc0_g, bnc0_b, Wc0, bc0,
           bnc1_g, bnc1_b, Wc1, bc1,
           bnc2_g, bnc2_b, Wc2, bc2,
           bnfc0_g, bnfc0_b, W_l0, b_l0,
           bn_h_g, bn_h_b, W_cls, b_cls):
    B, N, F = x.shape
    H = W_feat.shape[1]
    C = W_cls.shape[1]
    row = lambda a: a.reshape(1, -1)
    n_in = 26
    specs = [pl.BlockSpec(memory_space=pl.MemorySpace.ANY) if i == 1
             else pl.BlockSpec(memory_space=pltpu.MemorySpace.VMEM)
             for i in range(n_in)]
    return pl.pallas_call(
        _fwd_kernel,
        out_shape=jax.ShapeDtypeStruct((B, C), jnp.float32),
        in_specs=specs,
        scratch_shapes=[pltpu.VMEM((B * N, H), jnp.float32),
                        pltpu.VMEM((B, N, N), jnp.float32),
                        pltpu.SemaphoreType.DMA],
    )(x.reshape(B * N, F), adj,
      row(bn_feat_g), row(bn_feat_b), W_feat, row(b_feat),
      row(bnc0_g), row(bnc0_b), Wc0, row(bc0),
      row(bnc1_g), row(bnc1_b), Wc1, row(bc1),
      row(bnc2_g), row(bnc2_b), Wc2, row(bc2),
      row(bnfc0_g), row(bnfc0_b), W_l0, row(b_l0),
      row(bn_h_g), row(bn_h_b), W_cls, row(b_cls))
